# baseline (speedup 1.0000x reference)
import functools

import jax
import jax.numpy as jnp
from jax import lax
from jax.experimental import pallas as pl
from jax.experimental.pallas import tpu as pltpu
from jax.experimental.pallas import tpu_sc as plsc

R = 16
N = 2048
L = 262144

NUM_CORES = 2
NUM_SUBCORES = 16
NUM_WORKERS = NUM_CORES * NUM_SUBCORES
B = L // NUM_WORKERS
LANES = 16
STEPS = B // LANES


def _sc_body(table_hbm, e1_hbm, rel_hbm, e2_hbm, lab_hbm,
             outp_hbm, outn_hbm,
             e1_v, rel_v, e2_v, lab_v, idx_v, vals_v, accp_v, accn_v, sem):
    wid = lax.axis_index("s") * NUM_CORES + lax.axis_index("c")
    base = wid * B

    pltpu.sync_copy(e1_hbm.at[pl.ds(base, B)], e1_v)
    pltpu.sync_copy(rel_hbm.at[pl.ds(base, B)], rel_v)
    pltpu.sync_copy(e2_hbm.at[pl.ds(base, B)], e2_v)
    pltpu.sync_copy(lab_hbm.at[pl.ds(base, B)], lab_v)

    def idx_body(i, carry):
        off = i * LANES
        flat = (rel_v[pl.ds(off, LANES)] * (N * N)
                + e1_v[pl.ds(off, LANES)] * N
                + e2_v[pl.ds(off, LANES)])
        idx_v[pl.ds(off, LANES)] = flat
        return carry

    lax.fori_loop(0, STEPS, idx_body, 0)

    pltpu.async_copy(table_hbm.at[idx_v], vals_v, sem).wait()

    def acc_body(i, carry):
        accp, accn = carry
        off = i * LANES
        x = vals_v[pl.ds(off, LANES)]
        lb = lab_v[pl.ds(off, LANES)]
        sf = (2 * lb - 1).astype(jnp.float32)
        p = 1.0 / (1.0 + jnp.exp(-(sf * x)))
        return accp + p, accn + (1.0 - sf) * 0.5

    zeros = jnp.zeros((LANES,), jnp.float32)
    accp, accn = lax.fori_loop(0, STEPS, acc_body, (zeros, zeros))

    accp_v[...] = accp
    accn_v[...] = accn
    pltpu.sync_copy(accp_v, outp_hbm.at[wid])
    pltpu.sync_copy(accn_v, outn_hbm.at[wid])


@functools.partial(
    pl.kernel,
    out_type=(
        jax.ShapeDtypeStruct((NUM_WORKERS, LANES), jnp.float32),
        jax.ShapeDtypeStruct((NUM_WORKERS, LANES), jnp.float32),
    ),
    mesh=plsc.VectorSubcoreMesh(
        core_axis_name="c", subcore_axis_name="s",
        num_cores=NUM_CORES, num_subcores=NUM_SUBCORES,
    ),
    scratch_types=[
        pltpu.VMEM((B,), jnp.int32),
        pltpu.VMEM((B,), jnp.int32),
        pltpu.VMEM((B,), jnp.int32),
        pltpu.VMEM((B,), jnp.int32),
        pltpu.VMEM((B,), jnp.int32),
        pltpu.VMEM((B,), jnp.float32),
        pltpu.VMEM((LANES,), jnp.float32),
        pltpu.VMEM((LANES,), jnp.float32),
        pltpu.SemaphoreType.DMA,
    ],
)
def _sc_loss(*refs):
    _sc_body(*refs)


def kernel(predicted_values, labels):
    table = predicted_values.reshape(-1)
    lab32 = labels.astype(jnp.int32)
    e1 = lab32[:, 0]
    rel = lab32[:, 1]
    e2 = lab32[:, 2]
    lb = lab32[:, 3]
    partial_p, partial_n = _sc_loss(table, e1, rel, e2, lb)
    sum_p = jnp.sum(partial_p)
    neg = jnp.sum(partial_n)
    loss = (-1.0 / ((1.0 + neg) * jnp.float32(L))) * sum_p
    return jnp.reshape(loss, (1,)).astype(jnp.float32)
